# baseline (speedup 1.0000x reference)
import functools

import jax
import jax.numpy as jnp
from jax import lax
from jax.experimental import pallas as pl
from jax.experimental.pallas import tpu as pltpu
from jax.experimental.pallas import tpu_sc as plsc


def _sc_gather4(user_ids, item_ids, gu_t, gi_t, mu_t, mi_t):
    B = user_ids.shape[0]
    D = gu_t.shape[1]
    info = plsc.get_sparse_core_info()
    nc, ns = info.num_cores, info.num_subcores
    nw = nc * ns
    b_per_w = B // nw
    assert B % nw == 0 and b_per_w % 8 == 0

    mesh = plsc.VectorSubcoreMesh(core_axis_name="c", subcore_axis_name="s")
    fs = jax.ShapeDtypeStruct((B, D), jnp.float32)

    @functools.partial(
        pl.kernel,
        mesh=mesh,
        compiler_params=pltpu.CompilerParams(use_tc_tiling_on_sc=False),
        out_type=[fs, fs, fs, fs],
        scratch_types=[
            pltpu.VMEM((b_per_w,), jnp.int32),
            pltpu.VMEM((b_per_w,), jnp.int32),
            pltpu.VMEM((b_per_w, D), jnp.float32),
            pltpu.VMEM((b_per_w, D), jnp.float32),
            pltpu.VMEM((b_per_w, D), jnp.float32),
            pltpu.VMEM((b_per_w, D), jnp.float32),
            pltpu.SemaphoreType.DMA,
        ],
    )
    def gather_kernel(uid_h, iid_h, gut_h, git_h, mut_h, mit_h,
                      gu_o, gi_o, mu_o, mi_o,
                      uidx, iidx, rgu, rgi, rmu, rmi, sem):
        wid = lax.axis_index("s") * nc + lax.axis_index("c")
        base = wid * b_per_w
        pltpu.sync_copy(uid_h.at[pl.ds(base, b_per_w)], uidx)
        pltpu.sync_copy(iid_h.at[pl.ds(base, b_per_w)], iidx)
        c1 = pltpu.async_copy(gut_h.at[uidx], rgu, sem)
        c2 = pltpu.async_copy(git_h.at[iidx], rgi, sem)
        c3 = pltpu.async_copy(mut_h.at[uidx], rmu, sem)
        c4 = pltpu.async_copy(mit_h.at[iidx], rmi, sem)
        c1.wait()
        c2.wait()
        c3.wait()
        c4.wait()
        pltpu.sync_copy(rgu, gu_o.at[pl.ds(base, b_per_w)])
        pltpu.sync_copy(rgi, gi_o.at[pl.ds(base, b_per_w)])
        pltpu.sync_copy(rmu, mu_o.at[pl.ds(base, b_per_w)])
        pltpu.sync_copy(rmi, mi_o.at[pl.ds(base, b_per_w)])

    return gather_kernel(user_ids, item_ids, gu_t, gi_t, mu_t, mi_t)


def _tc_mlp(gu, gi, mu, mi, W1, b1, W2, b2, W3, b3, WoT, bo):
    B = gu.shape[0]
    blk = 2048
    grid = B // blk
    d = gu.shape[1]

    def body(gu_r, gi_r, mu_r, mi_r, W1_r, b1_r, W2_r, b2_r, W3_r, b3_r,
             Wo_r, bo_r, out_r):
        h = jnp.concatenate([mu_r[...], mi_r[...]], axis=-1)
        h = jnp.maximum(
            jnp.dot(h, W1_r[...], preferred_element_type=jnp.float32)
            + b1_r[...], 0.0)
        h = jnp.maximum(
            jnp.dot(h, W2_r[...], preferred_element_type=jnp.float32)
            + b2_r[...], 0.0)
        h = jnp.maximum(
            jnp.dot(h, W3_r[...], preferred_element_type=jnp.float32)
            + b3_r[...], 0.0)
        gmf = gu_r[...] * gi_r[...]
        z = jnp.concatenate([gmf, h], axis=-1)
        logit = jnp.sum(z * Wo_r[...], axis=-1) + bo_r[0, 0]
        out_r[...] = 1.0 / (1.0 + jnp.exp(-logit))

    full = lambda a: pl.BlockSpec(a.shape, lambda i: (0,) * a.ndim)
    emb_spec = pl.BlockSpec((blk, d), lambda i: (i, 0))
    return pl.pallas_call(
        body,
        grid=(grid,),
        in_specs=[emb_spec, emb_spec, emb_spec, emb_spec,
                  full(W1), full(b1), full(W2), full(b2), full(W3), full(b3),
                  full(WoT), full(bo)],
        out_specs=pl.BlockSpec((blk,), lambda i: (i,)),
        out_shape=jax.ShapeDtypeStruct((B,), jnp.float32),
    )(gu, gi, mu, mi, W1, b1, W2, b2, W3, b3, WoT, bo)


def kernel(user_ids, item_ids, gmf_user_emb, gmf_item_emb, mlp_user_emb,
           mlp_item_emb, W1, b1, W2, b2, W3, b3, Wout, bout):
    user_ids = user_ids.astype(jnp.int32)
    item_ids = item_ids.astype(jnp.int32)
    gu, gi, mu, mi = _sc_gather4(user_ids, item_ids, gmf_user_emb,
                                 gmf_item_emb, mlp_user_emb, mlp_item_emb)
    return _tc_mlp(gu, gi, mu, mi,
                   W1, b1.reshape(1, -1),
                   W2, b2.reshape(1, -1),
                   W3, b3.reshape(1, -1),
                   Wout.reshape(1, -1), bout.reshape(1, 1))


# speedup vs baseline: 3.7836x; 3.7836x over previous
import functools

import jax
import jax.numpy as jnp
from jax import lax
from jax.experimental import pallas as pl
from jax.experimental.pallas import tpu as pltpu
from jax.experimental.pallas import tpu_sc as plsc

_RD = 4


def _sc_gather4(user_ids, item_ids, guT, muT, giT, miT):
    B = user_ids.shape[0]
    D, _ = guT.shape
    info = plsc.get_sparse_core_info()
    nc, ns = info.num_cores, info.num_subcores
    nw = nc * ns
    bpw = B // nw
    assert B % nw == 0 and bpw % 128 == 0

    mesh = plsc.VectorSubcoreMesh(core_axis_name="c", subcore_axis_name="s")
    fs = jax.ShapeDtypeStruct((D, B), jnp.float32)

    @functools.partial(
        pl.kernel,
        mesh=mesh,
        compiler_params=pltpu.CompilerParams(needs_layout_passes=False),
        out_type=[fs, fs, fs, fs],
        scratch_types=[
            pltpu.VMEM((bpw,), jnp.int32),
            pltpu.VMEM((bpw,), jnp.int32),
            pltpu.VMEM((4 * _RD, D, 128), jnp.float32),
            pltpu.VMEM((4, D, 128), jnp.float32),
            pltpu.SemaphoreType.DMA,
            pltpu.SemaphoreType.DMA,
            pltpu.SemaphoreType.DMA,
            pltpu.SemaphoreType.DMA,
            pltpu.SemaphoreType.DMA,
        ],
    )
    def gather_kernel(uid_h, iid_h, t0, t1, t2, t3, o0, o1, o2, o3,
                      uidx, iidx, ring, cols, s0, s1, s2, s3, colsem):
        wid = lax.axis_index("s") * nc + lax.axis_index("c")
        base = wid * bpw
        pltpu.sync_copy(uid_h.at[pl.ds(base, bpw)], uidx)
        pltpu.sync_copy(iid_h.at[pl.ds(base, bpw)], iidx)
        tabs = (t0, t1, t2, t3)
        outs = (o0, o1, o2, o3)
        qsems = (s0, s1, s2, s3)
        lane = lax.broadcasted_iota(jnp.int32, (16,), 0)
        krows = tuple(lane + 16 * r for r in range(D // 16))

        def get_scalar(ref, s):
            al = pl.multiple_of((s // 16) * 16, 16)
            vec = ref[pl.ds(al, 16)]
            return jnp.sum(jnp.where(lane == (s % 16), vec, 0))

        def fire(s, q):
            u = get_scalar(uidx, s)
            it = get_scalar(iidx, s)
            for t in range(4):
                v = u if t < 2 else it
                off = pl.multiple_of((v // 128) * 128, 128)
                pltpu.async_copy(
                    tabs[t].at[:, pl.ds(off, 128)],
                    ring.at[q * 4 + t],
                    qsems[q],
                )

        def drain(q):
            for t in range(4):
                pltpu.make_async_copy(
                    tabs[t].at[:, pl.ds(0, 128)],
                    ring.at[q * 4 + t],
                    qsems[q],
                ).wait()

        def extract(e, q):
            u = get_scalar(uidx, e)
            it = get_scalar(iidx, e)
            c = e % 128
            csplat = jnp.full((16,), c, jnp.int32)
            for t in range(4):
                v = u if t < 2 else it
                l = v % 128
                lsplat = jnp.full((16,), l, jnp.int32)
                src = ring.at[q * 4 + t]
                for kr in krows:
                    g = plsc.load_gather(src, [kr, lsplat])
                    plsc.store_scatter(cols.at[t], [kr, csplat], g)

            @pl.when(c == 127)
            def _flush():
                ooff = pl.multiple_of(base + (e // 128) * 128, 128)
                for t in range(4):
                    pltpu.async_copy(
                        cols.at[t], outs[t].at[:, pl.ds(ooff, 128)], colsem
                    )
                for t in range(4):
                    pltpu.make_async_copy(
                        cols.at[t], outs[t].at[:, pl.ds(ooff, 128)], colsem
                    ).wait()

        def body(qi, carry):
            for q in range(4):
                s = qi * 4 + q

                @pl.when(qi >= 1)
                def _d(q=q, s=s):
                    drain(q)
                    extract(s - 4, q)

                fire(s, q)
            return carry

        lax.fori_loop(0, bpw // 4, body, 0)
        for q in range(4):
            drain(q)
            extract(bpw - 4 + q, q)

    return gather_kernel(user_ids, item_ids, guT, muT, giT, miT)


def _tc_mlp(guT, giT, muT, miT, W1a, W1b, b1, W2, b2, W3, b3, wg, wh, bo):
    B = guT.shape[1]
    blk = 2048
    grid = B // blk
    d = guT.shape[0]
    dn = (((0,), (0,)), ((), ()))

    def body(gu_r, gi_r, mu_r, mi_r, W1a_r, W1b_r, b1_r, W2_r, b2_r, W3_r,
             b3_r, wg_r, wh_r, bo_r, out_r):
        h = jnp.maximum(
            lax.dot_general(mu_r[...], W1a_r[...], dn,
                            preferred_element_type=jnp.float32)
            + lax.dot_general(mi_r[...], W1b_r[...], dn,
                              preferred_element_type=jnp.float32)
            + b1_r[...], 0.0)
        h = jnp.maximum(
            jnp.dot(h, W2_r[...], preferred_element_type=jnp.float32)
            + b2_r[...], 0.0)
        h = jnp.maximum(
            jnp.dot(h, W3_r[...], preferred_element_type=jnp.float32)
            + b3_r[...], 0.0)
        gl = jnp.sum(gu_r[...] * gi_r[...] * wg_r[...], axis=0)
        logit = jnp.sum(h * wh_r[...], axis=1) + gl + bo_r[0, 0]
        out_r[...] = 1.0 / (1.0 + jnp.exp(-logit))

    full = lambda a: pl.BlockSpec(a.shape, lambda i: (0,) * a.ndim)
    emb_spec = pl.BlockSpec((d, blk), lambda i: (0, i))
    return pl.pallas_call(
        body,
        grid=(grid,),
        in_specs=[emb_spec, emb_spec, emb_spec, emb_spec,
                  full(W1a), full(W1b), full(b1), full(W2), full(b2),
                  full(W3), full(b3), full(wg), full(wh), full(bo)],
        out_specs=pl.BlockSpec((blk,), lambda i: (i,)),
        out_shape=jax.ShapeDtypeStruct((B,), jnp.float32),
    )(guT, giT, muT, miT, W1a, W1b, b1, W2, b2, W3, b3, wg, wh, bo)


def kernel(user_ids, item_ids, gmf_user_emb, gmf_item_emb, mlp_user_emb,
           mlp_item_emb, W1, b1, W2, b2, W3, b3, Wout, bout):
    user_ids = user_ids.astype(jnp.int32)
    item_ids = item_ids.astype(jnp.int32)
    guT, muT, giT, miT = _sc_gather4(
        user_ids, item_ids, gmf_user_emb.T, mlp_user_emb.T,
        gmf_item_emb.T, mlp_item_emb.T)
    return _tc_mlp(guT, giT, muT, miT,
                   W1[:32], W1[32:], b1.reshape(1, -1),
                   W2, b2.reshape(1, -1),
                   W3, b3.reshape(1, -1),
                   Wout[:32], Wout[32:].reshape(1, -1),
                   bout.reshape(1, 1))
